# baseline (speedup 1.0000x reference)
import jax
import jax.numpy as jnp
from jax.experimental import pallas as pl
from jax.experimental.pallas import tpu as pltpu

_BM = 480
_NS = 4
_H = _BM // _NS


def _sub(a, xf, xi, w, d_in):
    acc = jnp.dot(a.astype(jnp.bfloat16), xf, preferred_element_type=jnp.float32)
    deg = jnp.sum(a, axis=1, keepdims=True)
    agg = acc / jnp.clip(deg, 1e-6, None)
    return (jnp.dot(xi, w[:d_in], preferred_element_type=jnp.float32)
            + jnp.dot(agg, w[d_in:], preferred_element_type=jnp.float32))


def _body(xf_ref, a0, a1, a2, a3, xi_ref, w_ref, out_ref):
    d_in = xi_ref.shape[1]
    xf = xf_ref[...].astype(jnp.bfloat16)
    w = w_ref[...]
    for j, a_ref in enumerate((a0, a1, a2, a3)):
        sl = slice(j * _H, (j + 1) * _H)
        out_ref[sl, :] = _sub(a_ref[...], xf, xi_ref[sl, :], w, d_in)


def kernel(x, adj, W):
    n, d_in = x.shape
    d_out = W.shape[1]
    nm = pl.cdiv(n, _BM)

    def adj_spec(j):
        return pl.BlockSpec((_H, n), lambda i, j=j: (_NS * i + j, 0))

    return pl.pallas_call(
        _body,
        grid=(nm,),
        in_specs=[
            pl.BlockSpec((n, d_in), lambda i: (0, 0)),
            adj_spec(0), adj_spec(1), adj_spec(2), adj_spec(3),
            pl.BlockSpec((_BM, d_in), lambda i: (i, 0)),
            pl.BlockSpec((2 * d_in, d_out), lambda i: (0, 0)),
        ],
        out_specs=pl.BlockSpec((_BM, d_out), lambda i: (i, 0)),
        out_shape=jax.ShapeDtypeStruct((n, d_out), jnp.float32),
        compiler_params=pltpu.CompilerParams(
            dimension_semantics=("parallel",),
            vmem_limit_bytes=64 * 1024 * 1024,
        ),
    )(x, adj, adj, adj, adj, x, W)
